# baseline (speedup 1.0000x reference)
import jax
import jax.numpy as jnp
from jax import lax
from jax.experimental import pallas as pl
from jax.experimental.pallas import tpu as pltpu
from jax.experimental.pallas import tpu_sc as plsc

NC = 2
NS = 16
NW = NC * NS
L = 16
R = 16384
D = 2048
RPW = R // NW
VPW = RPW // L
CH = 16
NB = 2


def _body(out_hbm, m_hbm, src_hbm, mask_v, posm,
          buf0, buf1, sg0, sg1, ss0, ss1):
    bufs = (buf0, buf1)
    sgs = (sg0, sg1)
    sss = (ss0, ss1)

    cid = lax.axis_index("c")
    sid = lax.axis_index("s")
    wid = sid * NC + cid
    base_row = wid * RPW

    pltpu.sync_copy(m_hbm, mask_v)

    def _sum_row(j, acc):
        return acc + mask_v[pl.ds(j * L, L)]

    accv = lax.fori_loop(0, wid * VPW, _sum_row, jnp.zeros((L,), jnp.int32))
    nm_before = jnp.sum(accv)

    lane = lax.iota(jnp.int32, L)
    cnt_m = jnp.int32(0)
    for j in range(VPW):
        mv = mask_v[pl.ds(base_row + j * L, L)]
        rows = base_row + j * L + lane
        dm = cnt_m + plsc.cumsum(mv) - mv
        plsc.store_scatter(posm, [dm], rows, mask=mv > 0)
        cnt_m = cnt_m + jnp.sum(mv)

    def _drain(sem):
        pltpu.make_async_copy(src_hbm.at[pl.ds(0, CH)], buf0, sem).wait()

    n_m = (cnt_m + CH - 1) // CH

    def outer(c0, s):
        for b in range(NB):
            c = c0 + b

            @pl.when(c < n_m)
            def _g():
                @pl.when(c >= NB)
                def _w():
                    _drain(sss[b])
                k = c * CH + lane
                srcidx = jnp.where(k < cnt_m, nm_before + k,
                                   nm_before + cnt_m - 1)
                pltpu.async_copy(src_hbm.at[srcidx], bufs[b], sgs[b])
        for b in range(NB):
            c = c0 + b

            @pl.when(c < n_m)
            def _s():
                _drain(sgs[b])
                k = c * CH + lane
                pv = posm[pl.ds(c * CH, CH)]
                valid = k < cnt_m
                p_pad = jnp.max(jnp.where(valid, pv, jnp.int32(-2147483647)))
                tgt = jnp.where(valid, pv, p_pad)
                pltpu.async_copy(bufs[b], out_hbm.at[tgt], sss[b])
        return s

    lax.fori_loop(0, (n_m + NB - 1) // NB, lambda i, s: outer(i * NB, s), 0)
    for b in range(NB):
        @pl.when(b < n_m)
        def _d():
            _drain(sss[b])


_run = pl.kernel(
    _body,
    out_type=(),
    mesh=plsc.VectorSubcoreMesh(core_axis_name="c", subcore_axis_name="s"),
    scratch_types=[
        pltpu.VMEM((R,), jnp.int32),
        pltpu.VMEM((RPW,), jnp.int32),
        pltpu.VMEM((CH, D), jnp.float32),
        pltpu.VMEM((CH, D), jnp.float32),
        pltpu.SemaphoreType.DMA,
        pltpu.SemaphoreType.DMA,
        pltpu.SemaphoreType.DMA,
        pltpu.SemaphoreType.DMA,
    ],
    compiler_params=pltpu.CompilerParams(needs_layout_passes=False),
)


def kernel(inputs_embeds, mask_1d, source):
    m1d = mask_1d.reshape(R).astype(jnp.int32)
    out_ref = jax.new_ref(inputs_embeds.reshape(R, D))
    _run(out_ref, m1d, source)
    return out_ref[...].reshape(inputs_embeds.shape)


# speedup vs baseline: 1.2439x; 1.2439x over previous
import jax
import jax.numpy as jnp
from jax import lax
from jax.experimental import pallas as pl
from jax.experimental.pallas import tpu as pltpu
from jax.experimental.pallas import tpu_sc as plsc

NC = 2
NS = 16
NW = NC * NS
L = 16
R = 16384
D = 2048
RPW = R // NW
VPW = RPW // L
CH = 16
NB = 3
BIG = 2**30


def _body(x_hbm, m_hbm, src_hbm, out_hbm, mask_v, posm, posu,
          buf0, buf1, buf2, sg0, sg1, sg2, ss0, ss1, ss2):
    bufs = (buf0, buf1, buf2)
    sgs = (sg0, sg1, sg2)
    sss = (ss0, ss1, ss2)

    cid = lax.axis_index("c")
    sid = lax.axis_index("s")
    wid = sid * NC + cid
    base_row = wid * RPW

    pltpu.sync_copy(m_hbm, mask_v)

    def _sum_row(j, accs):
        a0, a1, a2, a3 = accs
        o = j * 4 * L
        return (a0 + mask_v[pl.ds(o, L)],
                a1 + mask_v[pl.ds(o + L, L)],
                a2 + mask_v[pl.ds(o + 2 * L, L)],
                a3 + mask_v[pl.ds(o + 3 * L, L)])

    z = jnp.zeros((L,), jnp.int32)
    a0, a1, a2, a3 = lax.fori_loop(0, wid * (VPW // 4), _sum_row,
                                   (z, z, z, z))
    nm_before = jnp.sum(a0 + a1 + a2 + a3)

    fill = BIG + jnp.zeros((L,), jnp.int32)
    for j in range(VPW):
        posu[pl.ds(j * L, L)] = fill

    lane = lax.iota(jnp.int32, L)
    cnt_m = jnp.int32(0)
    cnt_u = jnp.int32(0)
    for j in range(VPW):
        mv = mask_v[pl.ds(base_row + j * L, L)]
        rows = base_row + j * L + lane
        mb = mv > 0
        em = plsc.cumsum(mv) - mv
        dm = cnt_m + em
        du = cnt_u + lane - em
        plsc.store_scatter(posm, [dm], rows, mask=mb)
        plsc.store_scatter(posu, [du], rows, mask=jnp.logical_not(mb))
        cnt_m = cnt_m + jnp.sum(mv)
        cnt_u = (j + 1) * L - cnt_m

    p0u = jnp.min(posu[pl.ds(0, L)])

    def _drain(sem):
        pltpu.make_async_copy(x_hbm.at[pl.ds(0, CH)], buf0, sem).wait()

    def run_phase(n_ch, issue_gather, tidx):
        def outer(c0, s):
            for b in range(NB):
                c = c0 + b

                @pl.when(c < n_ch)
                def _g():
                    @pl.when(c >= NB)
                    def _w():
                        _drain(sss[b])
                    issue_gather(c, bufs[b], sgs[b])
            for b in range(NB):
                c = c0 + b

                @pl.when(c < n_ch)
                def _s():
                    _drain(sgs[b])
                    pltpu.async_copy(bufs[b], out_hbm.at[tidx(c)], sss[b])
            return s

        lax.fori_loop(0, (n_ch + NB - 1) // NB,
                      lambda i, s: outer(i * NB, s), 0)
        for b in range(NB):
            @pl.when(b < n_ch)
            def _d():
                _drain(sss[b])

    def u_idx(c):
        pv = posu[pl.ds(c * CH, CH)]
        k = c * CH + lane
        return jnp.where(k < cnt_u, pv, p0u)

    def u_gather(c, buf, sem):
        pltpu.async_copy(x_hbm.at[u_idx(c)], buf, sem)

    n_u = (cnt_u + CH - 1) // CH
    run_phase(n_u, u_gather, u_idx)

    def m_gather(c, buf, sem):
        k = c * CH + lane
        srcidx = jnp.where(k < cnt_m, nm_before + k, nm_before + cnt_m - 1)
        pltpu.async_copy(src_hbm.at[srcidx], buf, sem)

    def m_tidx(c):
        k = c * CH + lane
        pv = posm[pl.ds(c * CH, CH)]
        valid = k < cnt_m
        p_pad = jnp.max(jnp.where(valid, pv, jnp.int32(-2147483647)))
        return jnp.where(valid, pv, p_pad)

    n_m = (cnt_m + CH - 1) // CH
    run_phase(n_m, m_gather, m_tidx)


_run = pl.kernel(
    _body,
    out_type=jax.ShapeDtypeStruct((R, D), jnp.float32),
    mesh=plsc.VectorSubcoreMesh(core_axis_name="c", subcore_axis_name="s"),
    scratch_types=[
        pltpu.VMEM((R,), jnp.int32),
        pltpu.VMEM((RPW,), jnp.int32),
        pltpu.VMEM((RPW,), jnp.int32),
        pltpu.VMEM((CH, D), jnp.float32),
        pltpu.VMEM((CH, D), jnp.float32),
        pltpu.VMEM((CH, D), jnp.float32),
        pltpu.SemaphoreType.DMA,
        pltpu.SemaphoreType.DMA,
        pltpu.SemaphoreType.DMA,
        pltpu.SemaphoreType.DMA,
        pltpu.SemaphoreType.DMA,
        pltpu.SemaphoreType.DMA,
    ],
    compiler_params=pltpu.CompilerParams(needs_layout_passes=False),
)


def kernel(inputs_embeds, mask_1d, source):
    x2d = inputs_embeds.reshape(R, D)
    m1d = mask_1d.reshape(R).astype(jnp.int32)
    out = _run(x2d, m1d, source)
    return out.reshape(inputs_embeds.shape)


# speedup vs baseline: 1.2752x; 1.0251x over previous
import jax
import jax.numpy as jnp
from jax import lax
from jax.experimental import pallas as pl
from jax.experimental.pallas import tpu as pltpu
from jax.experimental.pallas import tpu_sc as plsc

NC = 2
NS = 16
NW = NC * NS
L = 16
R = 16384
D = 2048
RPW = R // NW
VPW = RPW // L
CH = 16
NB = 3
BIG = 2**30


def _body(x_hbm, m_hbm, src_hbm, out_hbm, mask_v, posm, posu,
          buf0, buf1, buf2, sg0, sg1, sg2, ss0, ss1, ss2):
    bufs = (buf0, buf1, buf2)
    sgs = (sg0, sg1, sg2)
    sss = (ss0, ss1, ss2)

    cid = lax.axis_index("c")
    sid = lax.axis_index("s")
    wid = sid * NC + cid
    base_row = wid * RPW

    pltpu.sync_copy(m_hbm, mask_v)

    def _sum_row(j, accs):
        a0, a1, a2, a3 = accs
        o = j * 4 * L
        return (a0 + mask_v[pl.ds(o, L)],
                a1 + mask_v[pl.ds(o + L, L)],
                a2 + mask_v[pl.ds(o + 2 * L, L)],
                a3 + mask_v[pl.ds(o + 3 * L, L)])

    z = jnp.zeros((L,), jnp.int32)
    a0, a1, a2, a3 = lax.fori_loop(0, wid * (VPW // 4), _sum_row,
                                   (z, z, z, z))
    nm_before = jnp.sum(a0 + a1 + a2 + a3)

    fill = BIG + jnp.zeros((L,), jnp.int32)
    for j in range(VPW):
        posu[pl.ds(j * L, L)] = fill

    lane = lax.iota(jnp.int32, L)
    cnt_m = jnp.int32(0)
    cnt_u = jnp.int32(0)
    for j in range(VPW):
        mv = mask_v[pl.ds(base_row + j * L, L)]
        rows = base_row + j * L + lane
        mb = mv > 0
        em = plsc.cumsum(mv) - mv
        dm = cnt_m + em
        du = cnt_u + lane - em
        plsc.store_scatter(posm, [dm], rows, mask=mb)
        plsc.store_scatter(posu, [du], rows, mask=jnp.logical_not(mb))
        cnt_m = cnt_m + jnp.sum(mv)
        cnt_u = (j + 1) * L - cnt_m

    p0u = jnp.min(posu[pl.ds(0, L)])

    def _drain(sem):
        pltpu.make_async_copy(x_hbm.at[pl.ds(0, CH)], buf0, sem).wait()

    n_u = (cnt_u + CH - 1) // CH
    n_m = (cnt_m + CH - 1) // CH
    n_t = n_u + n_m

    def u_idx(c):
        cs = jnp.maximum(jnp.minimum(c, n_u - 1), 0)
        pv = posu[pl.ds(cs * CH, CH)]
        k = cs * CH + lane
        return jnp.where(k < cnt_u, pv, p0u)

    def m_srcidx(c):
        cm = jnp.maximum(jnp.minimum(c - n_u, n_m - 1), 0)
        k = cm * CH + lane
        return jnp.where(k < cnt_m, nm_before + k, nm_before + cnt_m - 1)

    def m_tidx(c):
        cm = jnp.maximum(jnp.minimum(c - n_u, n_m - 1), 0)
        k = cm * CH + lane
        pv = posm[pl.ds(cm * CH, CH)]
        valid = k < cnt_m
        p_pad = jnp.max(jnp.where(valid, pv, jnp.int32(-2147483647)))
        return jnp.where(valid, pv, p_pad)

    def outer(c0, s):
        for b in range(NB):
            c = c0 + b

            @pl.when(c < n_t)
            def _g():
                @pl.when(c >= NB)
                def _w():
                    _drain(sss[b])

                @pl.when(c < n_u)
                def _gu():
                    pltpu.async_copy(x_hbm.at[u_idx(c)], bufs[b], sgs[b])

                @pl.when(c >= n_u)
                def _gm():
                    pltpu.async_copy(src_hbm.at[m_srcidx(c)], bufs[b],
                                     sgs[b])
        for b in range(NB):
            c = c0 + b

            @pl.when(c < n_t)
            def _s():
                _drain(sgs[b])
                tgt = jnp.where(c < n_u, u_idx(c), m_tidx(c))
                pltpu.async_copy(bufs[b], out_hbm.at[tgt], sss[b])
        return s

    lax.fori_loop(0, (n_t + NB - 1) // NB, lambda i, s: outer(i * NB, s), 0)
    for b in range(NB):
        @pl.when(b < n_t)
        def _d():
            _drain(sss[b])


_run = pl.kernel(
    _body,
    out_type=jax.ShapeDtypeStruct((R, D), jnp.float32),
    mesh=plsc.VectorSubcoreMesh(core_axis_name="c", subcore_axis_name="s"),
    scratch_types=[
        pltpu.VMEM((R,), jnp.int32),
        pltpu.VMEM((RPW,), jnp.int32),
        pltpu.VMEM((RPW,), jnp.int32),
        pltpu.VMEM((CH, D), jnp.float32),
        pltpu.VMEM((CH, D), jnp.float32),
        pltpu.VMEM((CH, D), jnp.float32),
        pltpu.SemaphoreType.DMA,
        pltpu.SemaphoreType.DMA,
        pltpu.SemaphoreType.DMA,
        pltpu.SemaphoreType.DMA,
        pltpu.SemaphoreType.DMA,
        pltpu.SemaphoreType.DMA,
    ],
    compiler_params=pltpu.CompilerParams(needs_layout_passes=False),
)


def kernel(inputs_embeds, mask_1d, source):
    x2d = inputs_embeds.reshape(R, D)
    m1d = mask_1d.reshape(R).astype(jnp.int32)
    out = _run(x2d, m1d, source)
    return out.reshape(inputs_embeds.shape)
